# baseline (speedup 1.0000x reference)
import jax
import jax.numpy as jnp
from jax.experimental import pallas as pl
from jax.experimental.pallas import tpu as pltpu

_UV = 10000
_DIN = 300
_HID = 64
_DOUT = 64
_B = 4096

_T = 200
_NT = _UV // _T
_G = 128
_NG = _B // _G
_S = 4


def _z1_body(embed_ref, w1_ref, b1_ref, o_ref):
    o_ref[...] = (
        jnp.dot(embed_ref[...], w1_ref[...], preferred_element_type=jnp.float32)
        + b1_ref[...]
    ).astype(jnp.bfloat16)


def _fused_body(tid_ref, adj_blk, z1_ref, w2_ref, b2_ref, adj_any, o_ref,
                z2_s, buf, sems):
    i = pl.program_id(0)

    def _issue(grp):
        slot = jax.lax.rem(grp, _S)
        for g in range(_G):
            pltpu.make_async_copy(
                adj_any.at[pl.ds(tid_ref[grp * _G + g], 1), :],
                buf.at[slot, pl.ds(g, 1), :],
                sems.at[slot],
            ).start()

    @pl.when(i < _NT)
    def _():
        a = adj_blk[...].astype(jnp.bfloat16)
        h = jnp.dot(a, z1_ref[...], preferred_element_type=jnp.float32)
        h = jnp.maximum(h, 0.0)
        z2_s[pl.ds(i * _T, _T), :] = jnp.dot(
            h, w2_ref[...], preferred_element_type=jnp.float32
        ).astype(jnp.bfloat16)

    @pl.when(i == _NT - 1)
    def _():
        for grp in range(min(_S - 1, _NG)):
            _issue(grp)

    @pl.when(i >= _NT)
    def _():
        j = i - _NT

        @pl.when(j + _S - 1 < _NG)
        def _():
            _issue(j + _S - 1)

        slot = jax.lax.rem(j, _S)
        pltpu.make_async_copy(
            adj_any.at[pl.ds(0, _G), :], buf.at[slot], sems.at[slot]
        ).wait()
        a = buf[slot].astype(jnp.bfloat16)
        o_ref[...] = (
            jnp.dot(a, z2_s[...], preferred_element_type=jnp.float32)
            + b2_ref[...]
        )


def kernel(X_tid, adj, embed, W1, b1, W2, b2):
    b1r = jnp.reshape(b1, (1, _HID))
    b2r = jnp.reshape(b2, (1, _DOUT))

    z1b = pl.pallas_call(
        _z1_body,
        out_shape=jax.ShapeDtypeStruct((_UV, _HID), jnp.bfloat16),
        in_specs=[
            pl.BlockSpec((_UV, _DIN), lambda: (0, 0)),
            pl.BlockSpec((_DIN, _HID), lambda: (0, 0)),
            pl.BlockSpec((1, _HID), lambda: (0, 0)),
        ],
        out_specs=pl.BlockSpec((_UV, _HID), lambda: (0, 0)),
    )(embed, W1, b1r)

    out = pl.pallas_call(
        _fused_body,
        grid_spec=pltpu.PrefetchScalarGridSpec(
            num_scalar_prefetch=1,
            grid=(_NT + _NG,),
            in_specs=[
                pl.BlockSpec((_T, _UV), lambda i, tid: (jnp.minimum(i, _NT - 1), 0)),
                pl.BlockSpec((_UV, _HID), lambda i, tid: (0, 0)),
                pl.BlockSpec((_HID, _DOUT), lambda i, tid: (0, 0)),
                pl.BlockSpec((1, _DOUT), lambda i, tid: (0, 0)),
                pl.BlockSpec(memory_space=pl.ANY),
            ],
            out_specs=pl.BlockSpec(
                (_G, _DOUT), lambda i, tid: (jnp.maximum(i - _NT, 0), 0)
            ),
            scratch_shapes=[
                pltpu.VMEM((_UV, _DOUT), jnp.bfloat16),
                pltpu.VMEM((_S, _G, _UV), jnp.int32),
                pltpu.SemaphoreType.DMA((_S,)),
            ],
        ),
        out_shape=jax.ShapeDtypeStruct((_B, _DOUT), jnp.float32),
    )(X_tid, adj, z1b, W2, b2r, adj)
    return out


# speedup vs baseline: 1.0103x; 1.0103x over previous
import jax
import jax.numpy as jnp
from jax.experimental import pallas as pl
from jax.experimental.pallas import tpu as pltpu

_UV = 10000
_DIN = 300
_HID = 64
_DOUT = 64
_B = 4096

_T = 200
_NT = _UV // _T
_G = 128
_NG = _B // _G
_S = 4


def _fused_body(tid_ref, adj_blk, embed_ref, w1_ref, b1_ref, w2_ref, b2_ref,
                adj_any, o_ref, z1_s, z2_s, buf, sems):
    i = pl.program_id(0)

    def _issue(grp):
        slot = jax.lax.rem(grp, _S)
        for g in range(_G):
            pltpu.make_async_copy(
                adj_any.at[pl.ds(tid_ref[grp * _G + g], 1), :],
                buf.at[slot, pl.ds(g, 1), :],
                sems.at[slot],
            ).start()

    @pl.when(i == 0)
    def _():
        z1_s[...] = (
            jnp.dot(embed_ref[...], w1_ref[...], preferred_element_type=jnp.float32)
            + b1_ref[...]
        ).astype(jnp.bfloat16)

    @pl.when(jnp.logical_and(i >= 1, i <= _NT))
    def _():
        t = i - 1
        a = adj_blk[...].astype(jnp.bfloat16)
        h = jnp.dot(a, z1_s[...], preferred_element_type=jnp.float32)
        h = jnp.maximum(h, 0.0)
        z2_s[pl.ds(t * _T, _T), :] = jnp.dot(
            h, w2_ref[...], preferred_element_type=jnp.float32
        ).astype(jnp.bfloat16)

    @pl.when(i == _NT)
    def _():
        for grp in range(min(_S - 1, _NG)):
            _issue(grp)

    @pl.when(i >= _NT + 1)
    def _():
        j = i - _NT - 1

        @pl.when(j + _S - 1 < _NG)
        def _():
            _issue(j + _S - 1)

        slot = jax.lax.rem(j, _S)
        pltpu.make_async_copy(
            adj_any.at[pl.ds(0, _G), :], buf.at[slot], sems.at[slot]
        ).wait()
        a = buf[slot].astype(jnp.bfloat16)
        o_ref[...] = (
            jnp.dot(a, z2_s[...], preferred_element_type=jnp.float32)
            + b2_ref[...]
        )


def kernel(X_tid, adj, embed, W1, b1, W2, b2):
    b1r = jnp.reshape(b1, (1, _HID))
    b2r = jnp.reshape(b2, (1, _DOUT))

    out = pl.pallas_call(
        _fused_body,
        grid_spec=pltpu.PrefetchScalarGridSpec(
            num_scalar_prefetch=1,
            grid=(1 + _NT + _NG,),
            in_specs=[
                pl.BlockSpec(
                    (_T, _UV),
                    lambda i, tid: (jnp.clip(i - 1, 0, _NT - 1), 0),
                ),
                pl.BlockSpec((_UV, _DIN), lambda i, tid: (0, 0)),
                pl.BlockSpec((_DIN, _HID), lambda i, tid: (0, 0)),
                pl.BlockSpec((1, _HID), lambda i, tid: (0, 0)),
                pl.BlockSpec((_HID, _DOUT), lambda i, tid: (0, 0)),
                pl.BlockSpec((1, _DOUT), lambda i, tid: (0, 0)),
                pl.BlockSpec(memory_space=pl.ANY),
            ],
            out_specs=pl.BlockSpec(
                (_G, _DOUT), lambda i, tid: (jnp.maximum(i - _NT - 1, 0), 0)
            ),
            scratch_shapes=[
                pltpu.VMEM((_UV, _HID), jnp.bfloat16),
                pltpu.VMEM((_UV, _DOUT), jnp.bfloat16),
                pltpu.VMEM((_S, _G, _UV), jnp.int32),
                pltpu.SemaphoreType.DMA((_S,)),
            ],
        ),
        out_shape=jax.ShapeDtypeStruct((_B, _DOUT), jnp.float32),
    )(X_tid, adj, embed, W1, b1r, W2, b2r, adj)
    return out


# speedup vs baseline: 1.0207x; 1.0103x over previous
import jax
import jax.numpy as jnp
from jax.experimental import pallas as pl
from jax.experimental.pallas import tpu as pltpu

_UV = 10000
_DIN = 300
_HID = 64
_DOUT = 64
_B = 4096

_T = 200
_NT = _UV // _T
_ST = 2
_CH = 25
_G = 128
_NG = _B // _G
_S = 3


def _fused_body(tid_ref, embed_ref, w1_ref, b1_ref, w2_ref, b2_ref,
                adj_any, o_ref, z1_s, z2_s, tbuf, buf, tsems, sems):
    i = pl.program_id(0)
    _C = _T // _CH

    def _issue_tile(t):
        slot = jax.lax.rem(t, _ST)
        for c in range(_CH):
            pltpu.make_async_copy(
                adj_any.at[pl.ds(t * _T + c * _C, _C), :],
                tbuf.at[slot, pl.ds(c * _C, _C), :],
                tsems.at[slot],
            ).start()

    def _issue(grp):
        slot = jax.lax.rem(grp, _S)
        for g in range(_G):
            pltpu.make_async_copy(
                adj_any.at[pl.ds(tid_ref[grp * _G + g], 1), :],
                buf.at[slot, pl.ds(g, 1), :],
                sems.at[slot],
            ).start()

    @pl.when(i == 0)
    def _():
        for t in range(min(_ST - 1, _NT)):
            _issue_tile(t)
        z1_s[...] = (
            jnp.dot(embed_ref[...], w1_ref[...], preferred_element_type=jnp.float32)
            + b1_ref[...]
        ).astype(jnp.bfloat16)

    @pl.when(jnp.logical_and(i >= 1, i <= _NT))
    def _():
        t = i - 1

        @pl.when(t + _ST - 1 < _NT)
        def _():
            _issue_tile(t + _ST - 1)

        slot = jax.lax.rem(t, _ST)
        pltpu.make_async_copy(
            adj_any.at[pl.ds(0, _T), :], tbuf.at[slot], tsems.at[slot]
        ).wait()
        a = tbuf[slot].astype(jnp.bfloat16)
        h = jnp.dot(a, z1_s[...], preferred_element_type=jnp.float32)
        h = jnp.maximum(h, 0.0)
        z2_s[pl.ds(t * _T, _T), :] = jnp.dot(
            h, w2_ref[...], preferred_element_type=jnp.float32
        ).astype(jnp.bfloat16)

    @pl.when(i == _NT)
    def _():
        for grp in range(min(_S - 1, _NG)):
            _issue(grp)

    @pl.when(i >= _NT + 1)
    def _():
        j = i - _NT - 1

        @pl.when(j + _S - 1 < _NG)
        def _():
            _issue(j + _S - 1)

        slot = jax.lax.rem(j, _S)
        pltpu.make_async_copy(
            adj_any.at[pl.ds(0, _G), :], buf.at[slot], sems.at[slot]
        ).wait()
        a = buf[slot].astype(jnp.bfloat16)
        o_ref[...] = (
            jnp.dot(a, z2_s[...], preferred_element_type=jnp.float32)
            + b2_ref[...]
        )


def kernel(X_tid, adj, embed, W1, b1, W2, b2):
    b1r = jnp.reshape(b1, (1, _HID))
    b2r = jnp.reshape(b2, (1, _DOUT))

    out = pl.pallas_call(
        _fused_body,
        grid_spec=pltpu.PrefetchScalarGridSpec(
            num_scalar_prefetch=1,
            grid=(1 + _NT + _NG,),
            in_specs=[
                pl.BlockSpec((_UV, _DIN), lambda i, tid: (0, 0)),
                pl.BlockSpec((_DIN, _HID), lambda i, tid: (0, 0)),
                pl.BlockSpec((1, _HID), lambda i, tid: (0, 0)),
                pl.BlockSpec((_HID, _DOUT), lambda i, tid: (0, 0)),
                pl.BlockSpec((1, _DOUT), lambda i, tid: (0, 0)),
                pl.BlockSpec(memory_space=pl.ANY),
            ],
            out_specs=pl.BlockSpec(
                (_G, _DOUT), lambda i, tid: (jnp.maximum(i - _NT - 1, 0), 0)
            ),
            scratch_shapes=[
                pltpu.VMEM((_UV, _HID), jnp.bfloat16),
                pltpu.VMEM((_UV, _DOUT), jnp.bfloat16),
                pltpu.VMEM((_ST, _T, _UV), jnp.int32),
                pltpu.VMEM((_S, _G, _UV), jnp.int32),
                pltpu.SemaphoreType.DMA((_ST,)),
                pltpu.SemaphoreType.DMA((_S,)),
            ],
        ),
        out_shape=jax.ShapeDtypeStruct((_B, _DOUT), jnp.float32),
    )(X_tid, embed, W1, b1r, W2, b2r, adj)
    return out


# speedup vs baseline: 1.0514x; 1.0301x over previous
import jax
import jax.numpy as jnp
from jax.experimental import pallas as pl
from jax.experimental.pallas import tpu as pltpu

_UV = 10000
_DIN = 300
_HID = 64
_DOUT = 64
_B = 4096

_T = 200
_NT = _UV // _T
_ST = 2
_CH = 5
_G = 128
_NG = _B // _G
_S = 3


def _fused_body(tid_ref, embed_ref, w1_ref, b1_ref, w2_ref, b2_ref,
                adj_any, o_ref, z1_s, z2_s, tbuf, buf, tsems, sems):
    i = pl.program_id(0)
    _C = _T // _CH

    def _issue_tile(t):
        slot = jax.lax.rem(t, _ST)
        for c in range(_CH):
            pltpu.make_async_copy(
                adj_any.at[pl.ds(t * _T + c * _C, _C), :],
                tbuf.at[slot, pl.ds(c * _C, _C), :],
                tsems.at[slot],
            ).start()

    def _issue(grp):
        slot = jax.lax.rem(grp, _S)
        for g in range(_G):
            pltpu.make_async_copy(
                adj_any.at[pl.ds(tid_ref[grp * _G + g], 1), :],
                buf.at[slot, pl.ds(g, 1), :],
                sems.at[slot],
            ).start()

    @pl.when(i == 0)
    def _():
        for t in range(min(_ST - 1, _NT)):
            _issue_tile(t)
        z1_s[...] = (
            jnp.dot(embed_ref[...], w1_ref[...], preferred_element_type=jnp.float32)
            + b1_ref[...]
        ).astype(jnp.bfloat16)

    @pl.when(jnp.logical_and(i >= 1, i <= _NT))
    def _():
        t = i - 1

        @pl.when(t + _ST - 1 < _NT)
        def _():
            _issue_tile(t + _ST - 1)

        slot = jax.lax.rem(t, _ST)
        pltpu.make_async_copy(
            adj_any.at[pl.ds(0, _T), :], tbuf.at[slot], tsems.at[slot]
        ).wait()
        a = tbuf[slot].astype(jnp.bfloat16)
        h = jnp.dot(a, z1_s[...], preferred_element_type=jnp.float32)
        h = jnp.maximum(h, 0.0)
        z2_s[pl.ds(t * _T, _T), :] = jnp.dot(
            h, w2_ref[...], preferred_element_type=jnp.float32
        ).astype(jnp.bfloat16)

    @pl.when(i == _NT)
    def _():
        for grp in range(min(_S - 1, _NG)):
            _issue(grp)

    @pl.when(i >= _NT + 1)
    def _():
        j = i - _NT - 1

        @pl.when(j + _S - 1 < _NG)
        def _():
            _issue(j + _S - 1)

        slot = jax.lax.rem(j, _S)
        pltpu.make_async_copy(
            adj_any.at[pl.ds(0, _G), :], buf.at[slot], sems.at[slot]
        ).wait()
        a = buf[slot].astype(jnp.bfloat16)
        o_ref[...] = (
            jnp.dot(a, z2_s[...], preferred_element_type=jnp.float32)
            + b2_ref[...]
        )


def kernel(X_tid, adj, embed, W1, b1, W2, b2):
    b1r = jnp.reshape(b1, (1, _HID))
    b2r = jnp.reshape(b2, (1, _DOUT))

    out = pl.pallas_call(
        _fused_body,
        grid_spec=pltpu.PrefetchScalarGridSpec(
            num_scalar_prefetch=1,
            grid=(1 + _NT + _NG,),
            in_specs=[
                pl.BlockSpec((_UV, _DIN), lambda i, tid: (0, 0)),
                pl.BlockSpec((_DIN, _HID), lambda i, tid: (0, 0)),
                pl.BlockSpec((1, _HID), lambda i, tid: (0, 0)),
                pl.BlockSpec((_HID, _DOUT), lambda i, tid: (0, 0)),
                pl.BlockSpec((1, _DOUT), lambda i, tid: (0, 0)),
                pl.BlockSpec(memory_space=pl.ANY),
            ],
            out_specs=pl.BlockSpec(
                (_G, _DOUT), lambda i, tid: (jnp.maximum(i - _NT - 1, 0), 0)
            ),
            scratch_shapes=[
                pltpu.VMEM((_UV, _HID), jnp.bfloat16),
                pltpu.VMEM((_UV, _DOUT), jnp.bfloat16),
                pltpu.VMEM((_ST, _T, _UV), jnp.int32),
                pltpu.VMEM((_S, _G, _UV), jnp.int32),
                pltpu.SemaphoreType.DMA((_ST,)),
                pltpu.SemaphoreType.DMA((_S,)),
            ],
        ),
        out_shape=jax.ShapeDtypeStruct((_B, _DOUT), jnp.float32),
    )(X_tid, embed, W1, b1r, W2, b2r, adj)
    return out
